# baseline (speedup 1.0000x reference)
import functools

import jax
import jax.numpy as jnp
from jax import lax
from jax.experimental import pallas as pl
from jax.experimental.pallas import tpu as pltpu
from jax.experimental.pallas import tpu_sc as plsc

_ROWS = 16384
_SEQ = 50
_DIM = 32

_NC = 2
_NS = 16
_NW = _NC * _NS
_CROWS = 32


def _make_sc_gather(rows):
    r_per_w = rows // _NW
    n_chunks = r_per_w // _CROWS

    @functools.partial(
        pl.kernel,
        out_type=jax.ShapeDtypeStruct((rows, _SEQ, _DIM), jnp.bfloat16),
        mesh=plsc.VectorSubcoreMesh(core_axis_name="c", subcore_axis_name="s"),
        scratch_types=[
            pltpu.VMEM((2, _CROWS, _SEQ), jnp.int32),
            pltpu.VMEM((2, _CROWS * _SEQ, _DIM), jnp.bfloat16),
            pltpu.SemaphoreType.DMA,
            pltpu.SemaphoreType.DMA,
            pltpu.SemaphoreType.DMA,
            pltpu.SemaphoreType.DMA,
        ],
        compiler_params=pltpu.CompilerParams(use_tc_tiling_on_sc=False),
    )
    def _sc_gather(table_hbm, idx_hbm, out_hbm, idx_v, rows_v, g0, g1, o0, o1):
        wid = lax.axis_index("s") * _NC + lax.axis_index("c")
        base = wid * r_per_w
        gsem = (g0, g1)
        osem = (o0, o1)

        def idx_load(i, b):
            pltpu.sync_copy(idx_hbm.at[pl.ds(base + i * _CROWS, _CROWS), :],
                            idx_v.at[b])

        def gather_start(i, b):
            return [
                pltpu.async_copy(table_hbm.at[idx_v.at[b, j]],
                                 rows_v.at[b, pl.ds(j * _SEQ, _SEQ), :],
                                 gsem[b])
                for j in range(_CROWS)
            ]

        def out_start(i, b):
            return [
                pltpu.async_copy(rows_v.at[b, pl.ds(j * _SEQ, _SEQ), :],
                                 out_hbm.at[base + i * _CROWS + j, :, :],
                                 osem[b])
                for j in range(_CROWS)
            ]

        idx_load(0, 0)
        gathers = {0: gather_start(0, 0)}
        outs = {}
        for i in range(n_chunks):
            b = i % 2
            b2 = (i + 1) % 2
            if i + 1 < n_chunks:
                if i >= 1:
                    for cp in outs.pop(i - 1):
                        cp.wait()
                idx_load(i + 1, b2)
                gathers[i + 1] = gather_start(i + 1, b2)
            for cp in gathers.pop(i):
                cp.wait()
            outs[i] = out_start(i, b)
        for cp in outs.pop(n_chunks - 2):
            cp.wait()
        for cp in outs.pop(n_chunks - 1):
            cp.wait()

    return _sc_gather


_NSPLIT = 4
_gather_part = _make_sc_gather(_ROWS // _NSPLIT)


def kernel(token_ids, embeddings):
    part = _ROWS // _NSPLIT
    table = embeddings.astype(jnp.bfloat16)
    outs = [
        _gather_part(table, token_ids[i * part:(i + 1) * part])
        for i in range(_NSPLIT)
    ]
    return jnp.concatenate(outs, axis=0).astype(jnp.float32)


# speedup vs baseline: 1.1816x; 1.1816x over previous
import functools

import jax
import jax.numpy as jnp
from jax import lax
from jax.experimental import pallas as pl
from jax.experimental.pallas import tpu as pltpu
from jax.experimental.pallas import tpu_sc as plsc

_ROWS = 16384
_SEQ = 50
_DIM = 32

_NC = 2
_NS = 16
_NW = _NC * _NS
_CROWS = 32


def _make_sc_gather(rows):
    r_per_w = rows // _NW
    n_chunks = r_per_w // _CROWS

    @functools.partial(
        pl.kernel,
        out_type=jax.ShapeDtypeStruct((rows, _SEQ, _DIM), jnp.float32),
        mesh=plsc.VectorSubcoreMesh(core_axis_name="c", subcore_axis_name="s"),
        scratch_types=[
            pltpu.VMEM((2, _CROWS, _SEQ), jnp.int32),
            pltpu.VMEM((2, _CROWS * _SEQ, _DIM), jnp.float32),
            pltpu.SemaphoreType.DMA,
            pltpu.SemaphoreType.DMA,
            pltpu.SemaphoreType.DMA,
            pltpu.SemaphoreType.DMA,
        ],
        compiler_params=pltpu.CompilerParams(use_tc_tiling_on_sc=False),
    )
    def _sc_gather(table_hbm, idx_hbm, out_hbm, idx_v, rows_v, g0, g1, o0, o1):
        wid = lax.axis_index("s") * _NC + lax.axis_index("c")
        base = wid * r_per_w
        gsem = (g0, g1)
        osem = (o0, o1)

        def idx_load(i, b):
            pltpu.sync_copy(idx_hbm.at[pl.ds(base + i * _CROWS, _CROWS), :],
                            idx_v.at[b])

        def gather_start(i, b):
            return [
                pltpu.async_copy(table_hbm.at[idx_v.at[b, j]],
                                 rows_v.at[b, pl.ds(j * _SEQ, _SEQ), :],
                                 gsem[b])
                for j in range(_CROWS)
            ]

        def out_start(i, b):
            return [
                pltpu.async_copy(rows_v.at[b, pl.ds(j * _SEQ, _SEQ), :],
                                 out_hbm.at[base + i * _CROWS + j, :, :],
                                 osem[b])
                for j in range(_CROWS)
            ]

        idx_load(0, 0)
        gathers = {0: gather_start(0, 0)}
        outs = {}
        for i in range(n_chunks):
            b = i % 2
            b2 = (i + 1) % 2
            if i + 1 < n_chunks:
                if i >= 1:
                    for cp in outs.pop(i - 1):
                        cp.wait()
                idx_load(i + 1, b2)
                gathers[i + 1] = gather_start(i + 1, b2)
            for cp in gathers.pop(i):
                cp.wait()
            outs[i] = out_start(i, b)
        for cp in outs.pop(n_chunks - 2):
            cp.wait()
        for cp in outs.pop(n_chunks - 1):
            cp.wait()

    return _sc_gather


_NSPLIT = 4
_gather_part = _make_sc_gather(_ROWS // _NSPLIT)


def kernel(token_ids, embeddings):
    part = _ROWS // _NSPLIT
    outs = [
        _gather_part(embeddings, token_ids[i * part:(i + 1) * part])
        for i in range(_NSPLIT)
    ]
    return jnp.concatenate(outs, axis=0)
